# baseline (speedup 1.0000x reference)
import functools
import math

import jax
import jax.numpy as jnp
from jax import lax
from jax.experimental import pallas as pl
from jax.experimental.pallas import tpu as pltpu
from jax.experimental.pallas import tpu_sc as plsc

NUM_ROADS = 512
N_LANE = 50000
N_ROAD = 50000
D = 128
N_PATH = 25000


_A_BLK = 5000
_A_GRID = N_ROAD // _A_BLK
_B_BLK = 5000
_B_GRID = N_LANE // _B_BLK
_SCALE = 1.0 / math.sqrt(D)


def _ab_body(road_ref, idx_ref, lane_ref, cnt_ref, sum_ref, sim_ref,
             soft_ref, acc_sum, invc_s):
    i = pl.program_id(0)

    @pl.when(i == 0)
    def _init():
        acc_sum[...] = jnp.zeros_like(acc_sum)

    @pl.when(i < _A_GRID)
    def _seg():
        idx = idx_ref[0]
        iota = lax.broadcasted_iota(jnp.int32, (NUM_ROADS, _A_BLK), 0)
        onehot_t = (iota == idx).astype(jnp.bfloat16)
        acc_sum[...] += lax.dot_general(
            onehot_t, road_ref[...].astype(jnp.bfloat16),
            (((1,), (0,)), ((), ())), preferred_element_type=jnp.float32)

    @pl.when(i == _A_GRID)
    def _mean():
        cnt = cnt_ref[...]
        c = cnt[0:1, :] + cnt[1:2, :]
        invc_s[...] = _SCALE / jnp.clip(c, 1.0, None)
        sum_ref[...] = acc_sum[...]

    @pl.when(i >= _A_GRID)
    def _sim():
        sim = lax.dot_general(
            lane_ref[...], acc_sum[...], (((1,), (1,)), ((), ())),
            preferred_element_type=jnp.float32) * invc_s[...]
        sim_ref[...] = sim
        mx = jnp.max(sim, axis=1, keepdims=True)
        e = jnp.exp(sim - mx)
        s = jnp.sum(e, axis=1, keepdims=True)
        soft_ref[...] = e / s


def _seg_sim_tc(road_feat, road_idx, lane_feat, road_cnt):
    idx3 = road_idx.reshape(_A_GRID, 1, _A_BLK)
    grid = _A_GRID + _B_GRID
    return pl.pallas_call(
        _ab_body,
        grid=(grid,),
        in_specs=[
            pl.BlockSpec((_A_BLK, D), lambda i: (jnp.minimum(i, _A_GRID - 1), 0)),
            pl.BlockSpec((1, 1, _A_BLK),
                         lambda i: (jnp.minimum(i, _A_GRID - 1), 0, 0)),
            pl.BlockSpec((_B_BLK, D),
                         lambda i: (jnp.maximum(i - _A_GRID, 0), 0)),
            pl.BlockSpec((2, NUM_ROADS), lambda i: (0, 0)),
        ],
        out_specs=[
            pl.BlockSpec((NUM_ROADS, D), lambda i: (0, 0)),
            pl.BlockSpec((_B_BLK, NUM_ROADS),
                         lambda i: (jnp.maximum(i - _A_GRID, 0), 0)),
            pl.BlockSpec((_B_BLK, NUM_ROADS),
                         lambda i: (jnp.maximum(i - _A_GRID, 0), 0)),
        ],
        out_shape=[
            jax.ShapeDtypeStruct((NUM_ROADS, D), jnp.float32),
            jax.ShapeDtypeStruct((N_LANE, NUM_ROADS), jnp.float32),
            jax.ShapeDtypeStruct((N_LANE, NUM_ROADS), jnp.float32),
        ],
        scratch_shapes=[
            pltpu.VMEM((NUM_ROADS, D), jnp.float32),
            pltpu.VMEM((1, NUM_ROADS), jnp.float32),
        ],
    )(road_feat, idx3, lane_feat, road_cnt)


_NC = 2
_NS = 16
_NW = _NC * _NS
_BPW = 784
_RPW = 1568
_RTAIL = 176


def _path_gather_sc(lane_feat, path_inverse, gt, road_idx, wmat, zmat):
    mesh = plsc.VectorSubcoreMesh(core_axis_name="c", subcore_axis_name="s")

    @functools.partial(
        pl.kernel,
        mesh=mesh,
        out_type=[
            jax.ShapeDtypeStruct((N_PATH, D), jnp.float32),
            jax.ShapeDtypeStruct((N_PATH,), jnp.int32),
            jax.ShapeDtypeStruct((_NC, NUM_ROADS), jnp.float32),
        ],
        scratch_types=[
            pltpu.VMEM((_BPW,), jnp.int32),
            pltpu.VMEM((_BPW, D), jnp.float32),
            pltpu.VMEM((_BPW,), jnp.int32),
            pltpu.VMEM((_RPW,), jnp.int32),
            pltpu.VMEM((_RPW,), jnp.float32),
            pltpu.VMEM((NUM_ROADS,), jnp.float32),
            pltpu.VMEM_SHARED((NUM_ROADS,), jnp.float32),
            pltpu.SemaphoreType.DMA,
        ],
    )
    def k(lane_hbm, pinv_hbm, gt_hbm, ridx_hbm, wmat_hbm, zmat_hbm,
          plane_hbm, pgt_hbm, cnt_hbm,
          idx_v, rows_v, gt_v, ridx_v, w_v, ztmp_v, shared_cnt, sem):
        cid = lax.axis_index("c")
        sid = lax.axis_index("s")
        wid = sid * _NC + cid
        base = jnp.minimum(wid * _BPW, N_PATH - _BPW)
        pltpu.sync_copy(pinv_hbm.at[pl.ds(base, _BPW)], idx_v)
        gt_cp = pltpu.async_copy(gt_hbm.at[idx_v], gt_v, sem)
        rows_cp = pltpu.async_copy(lane_hbm.at[idx_v], rows_v, sem)

        @pl.when(sid == 0)
        def _zero():
            pltpu.sync_copy(zmat_hbm, ztmp_v)
            pltpu.sync_copy(ztmp_v, shared_cnt)

        rbase = jnp.minimum(wid * _RPW, N_ROAD - _RPW)
        pltpu.sync_copy(ridx_hbm.at[pl.ds(rbase, _RPW)], ridx_v)
        wrow = jnp.where(wid == _NW - 1, 1, 0)
        pltpu.sync_copy(wmat_hbm.at[wrow], w_v)
        plsc.subcore_barrier()
        pltpu.sync_copy(w_v, shared_cnt.at[ridx_v], add=True)
        plsc.subcore_barrier()

        @pl.when(sid == 0)
        def _export():
            pltpu.sync_copy(shared_cnt, ztmp_v)
            pltpu.sync_copy(ztmp_v, cnt_hbm.at[cid])

        gt_cp.wait()
        rows_cp.wait()
        pltpu.sync_copy(gt_v, pgt_hbm.at[pl.ds(base, _BPW)])
        pltpu.sync_copy(rows_v, plane_hbm.at[pl.ds(base, _BPW)])

    return k(lane_feat, path_inverse, gt, road_idx, wmat, zmat)


_P_BLK = 5000
_P_GRID = N_PATH // _P_BLK


def _psim_body(plane_ref, sum_ref, cnt_ref, out_ref):
    cnt = cnt_ref[...]
    invc = _SCALE / jnp.clip(cnt[0:1, :] + cnt[1:2, :], 1.0, None)
    out_ref[...] = lax.dot_general(
        plane_ref[...], sum_ref[...], (((1,), (1,)), ((), ())),
        preferred_element_type=jnp.float32) * invc


def _psim_tc(path_lane, road_sum, road_cnt):
    return pl.pallas_call(
        _psim_body,
        grid=(_P_GRID,),
        in_specs=[
            pl.BlockSpec((_P_BLK, D), lambda i: (i, 0)),
            pl.BlockSpec((NUM_ROADS, D), lambda i: (0, 0)),
            pl.BlockSpec((2, NUM_ROADS), lambda i: (0, 0)),
        ],
        out_specs=pl.BlockSpec((_P_BLK, NUM_ROADS), lambda i: (i, 0)),
        out_shape=jax.ShapeDtypeStruct((N_PATH, NUM_ROADS), jnp.float32),
    )(path_lane, road_sum, road_cnt)


def kernel(lane_feat, road_feat, road_idx, path_inverse, gt):
    w0 = jnp.ones((_RPW,), jnp.float32)
    wmat = jnp.stack([w0, w0.at[:_RTAIL].set(0.0)])
    zmat = jnp.zeros((NUM_ROADS,), jnp.float32)
    path_lane, path_gt, road_cnt = _path_gather_sc(
        lane_feat, path_inverse, gt, road_idx, wmat, zmat)
    road_sum, sim, sim_softmax = _seg_sim_tc(road_feat, road_idx, lane_feat,
                                             road_cnt)
    path_sim = _psim_tc(path_lane, road_sum, road_cnt)
    return sim, sim_softmax, path_sim, path_gt


# speedup vs baseline: 1.0647x; 1.0647x over previous
import functools
import math

import jax
import jax.numpy as jnp
from jax import lax
from jax.experimental import pallas as pl
from jax.experimental.pallas import tpu as pltpu
from jax.experimental.pallas import tpu_sc as plsc

NUM_ROADS = 512
N_LANE = 50000
N_ROAD = 50000
D = 128
N_PATH = 25000


_A_BLK = 5000
_A_GRID = N_ROAD // _A_BLK
_B_BLK = 5000
_B_GRID = N_LANE // _B_BLK
_SCALE = 1.0 / math.sqrt(D)


def _ab_body(road_ref, idx_ref, lane_ref, mean_ref, sim_ref, soft_ref,
             acc_sum, acc_cnt, mean_s):
    i = pl.program_id(0)

    @pl.when(i == 0)
    def _init():
        acc_sum[...] = jnp.zeros_like(acc_sum)
        acc_cnt[...] = jnp.zeros_like(acc_cnt)

    @pl.when(i < _A_GRID)
    def _seg():
        idx = idx_ref[0]
        iota = lax.broadcasted_iota(jnp.int32, (NUM_ROADS, _A_BLK), 0)
        mask = iota == idx
        onehot_t = mask.astype(jnp.bfloat16)
        acc_sum[...] += lax.dot_general(
            onehot_t, road_ref[...].astype(jnp.bfloat16),
            (((1,), (0,)), ((), ())), preferred_element_type=jnp.float32)
        cnt = jnp.sum(mask.astype(jnp.float32), axis=1, keepdims=True)
        acc_cnt[...] += jnp.broadcast_to(cnt, acc_cnt.shape)

    @pl.when(i == _A_GRID)
    def _mean():
        m = acc_sum[...] / jnp.clip(acc_cnt[:, :1], 1.0, None)
        mean_s[...] = m
        mean_ref[...] = m

    @pl.when(i >= _A_GRID)
    def _sim():
        sim = lax.dot_general(
            lane_ref[...], mean_s[...], (((1,), (1,)), ((), ())),
            preferred_element_type=jnp.float32) * _SCALE
        sim_ref[...] = sim
        mx = jnp.max(sim, axis=1, keepdims=True)
        e = jnp.exp(sim - mx)
        s = jnp.sum(e, axis=1, keepdims=True)
        soft_ref[...] = e / s


def _seg_sim_tc(road_feat, road_idx, lane_feat):
    idx3 = road_idx.reshape(_A_GRID, 1, _A_BLK)
    grid = _A_GRID + _B_GRID
    return pl.pallas_call(
        _ab_body,
        grid=(grid,),
        in_specs=[
            pl.BlockSpec((_A_BLK, D), lambda i: (jnp.minimum(i, _A_GRID - 1), 0)),
            pl.BlockSpec((1, 1, _A_BLK),
                         lambda i: (jnp.minimum(i, _A_GRID - 1), 0, 0)),
            pl.BlockSpec((_B_BLK, D),
                         lambda i: (jnp.maximum(i - _A_GRID, 0), 0)),
        ],
        out_specs=[
            pl.BlockSpec((NUM_ROADS, D), lambda i: (0, 0)),
            pl.BlockSpec((_B_BLK, NUM_ROADS),
                         lambda i: (jnp.maximum(i - _A_GRID, 0), 0)),
            pl.BlockSpec((_B_BLK, NUM_ROADS),
                         lambda i: (jnp.maximum(i - _A_GRID, 0), 0)),
        ],
        out_shape=[
            jax.ShapeDtypeStruct((NUM_ROADS, D), jnp.float32),
            jax.ShapeDtypeStruct((N_LANE, NUM_ROADS), jnp.float32),
            jax.ShapeDtypeStruct((N_LANE, NUM_ROADS), jnp.float32),
        ],
        scratch_shapes=[
            pltpu.VMEM((NUM_ROADS, D), jnp.float32),
            pltpu.VMEM((NUM_ROADS, 8), jnp.float32),
            pltpu.VMEM((NUM_ROADS, D), jnp.float32),
        ],
    )(road_feat, idx3, lane_feat)


_NC = 2
_NS = 16
_NW = _NC * _NS
_BPW = 784


def _path_gather_sc(lane_feat, path_inverse, gt):
    mesh = plsc.VectorSubcoreMesh(core_axis_name="c", subcore_axis_name="s")

    @functools.partial(
        pl.kernel,
        mesh=mesh,
        out_type=[
            jax.ShapeDtypeStruct((N_PATH, D), jnp.float32),
            jax.ShapeDtypeStruct((N_PATH,), jnp.int32),
        ],
        scratch_types=[
            pltpu.VMEM((_BPW,), jnp.int32),
            pltpu.VMEM((_BPW, D), jnp.float32),
            pltpu.VMEM((_BPW,), jnp.int32),
            pltpu.SemaphoreType.DMA,
        ],
    )
    def k(lane_hbm, pinv_hbm, gt_hbm, plane_hbm, pgt_hbm, idx_v, rows_v,
          gt_v, sem):
        wid = lax.axis_index("s") * _NC + lax.axis_index("c")
        base = jnp.minimum(wid * _BPW, N_PATH - _BPW)
        pltpu.sync_copy(pinv_hbm.at[pl.ds(base, _BPW)], idx_v)
        gt_cp = pltpu.async_copy(gt_hbm.at[idx_v], gt_v, sem)
        rows_cp = pltpu.async_copy(lane_hbm.at[idx_v], rows_v, sem)
        gt_cp.wait()
        rows_cp.wait()
        pltpu.sync_copy(gt_v, pgt_hbm.at[pl.ds(base, _BPW)])
        pltpu.sync_copy(rows_v, plane_hbm.at[pl.ds(base, _BPW)])

    return k(lane_feat, path_inverse, gt)


_P_BLK = 5000
_P_GRID = N_PATH // _P_BLK


def _psim_body(plane_ref, mean_ref, out_ref):
    out_ref[...] = lax.dot_general(
        plane_ref[...], mean_ref[...], (((1,), (1,)), ((), ())),
        preferred_element_type=jnp.float32) * _SCALE


def _psim_tc(path_lane, road_mean):
    return pl.pallas_call(
        _psim_body,
        grid=(_P_GRID,),
        in_specs=[
            pl.BlockSpec((_P_BLK, D), lambda i: (i, 0)),
            pl.BlockSpec((NUM_ROADS, D), lambda i: (0, 0)),
        ],
        out_specs=pl.BlockSpec((_P_BLK, NUM_ROADS), lambda i: (i, 0)),
        out_shape=jax.ShapeDtypeStruct((N_PATH, NUM_ROADS), jnp.float32),
    )(path_lane, road_mean)


def kernel(lane_feat, road_feat, road_idx, path_inverse, gt):
    path_lane, path_gt = _path_gather_sc(lane_feat, path_inverse, gt)
    road_mean, sim, sim_softmax = _seg_sim_tc(road_feat, road_idx, lane_feat)
    path_sim = _psim_tc(path_lane, road_mean)
    return sim, sim_softmax, path_sim, path_gt
